# baseline (speedup 1.0000x reference)
import functools

import jax
import jax.numpy as jnp
from jax import lax
from jax.experimental import pallas as pl
from jax.experimental.pallas import tpu as pltpu
from jax.experimental.pallas import tpu_sc as plsc

_VOCAB = 1000
_DPAD = 1024
_NC = 2
_NS = 16
_NW = _NC * _NS
_CHUNK = 24
_NBUF = 2
_REM = 16


def _sc_gather(idx_flat, table_p):
    n = idx_flat.shape[0]
    b_per_w = n // _NW
    n_chunks = (b_per_w - _REM) // _CHUNK
    mesh = plsc.VectorSubcoreMesh(core_axis_name="c", subcore_axis_name="s")

    @functools.partial(
        pl.kernel,
        mesh=mesh,
        compiler_params=pltpu.CompilerParams(use_tc_tiling_on_sc=False),
        out_type=jax.ShapeDtypeStruct((n, _DPAD), jnp.float32),
        scratch_types=[
            pltpu.VMEM((b_per_w,), jnp.int32),
            pltpu.VMEM((_NBUF, _CHUNK, _DPAD), jnp.float32),
            pltpu.VMEM_SHARED((_VOCAB, _DPAD), jnp.float32),
            pltpu.SemaphoreType.DMA,
            pltpu.SemaphoreType.DMA,
        ],
    )
    def k(idx_hbm, table_hbm, out_hbm, idx_v, gbuf, table_sp, gs0, gs1):
        gsems = [gs0, gs1]
        cid = lax.axis_index("c")
        sid = lax.axis_index("s")
        wid = sid * _NC + cid
        base = wid * b_per_w

        @pl.when(sid < 15)
        def _():
            pltpu.sync_copy(
                table_hbm.at[pl.ds(sid * 64, 64)],
                table_sp.at[pl.ds(sid * 64, 64)],
            )

        @pl.when(sid == 15)
        def _():
            pltpu.sync_copy(
                table_hbm.at[pl.ds(960, _VOCAB - 960)],
                table_sp.at[pl.ds(960, _VOCAB - 960)],
            )

        pltpu.sync_copy(idx_hbm.at[pl.ds(base, b_per_w)], idx_v)
        plsc.subcore_barrier()

        def start_g(c, b, nrows=_CHUNK):
            pltpu.async_copy(
                table_sp.at[idx_v.at[pl.ds(c * _CHUNK, nrows)]],
                gbuf.at[b, pl.ds(0, nrows)],
                gsems[b],
            )

        def wait_g(c, b, nrows=_CHUNK):
            pltpu.make_async_copy(
                table_sp.at[idx_v.at[pl.ds(c * _CHUNK, nrows)]],
                gbuf.at[b, pl.ds(0, nrows)],
                gsems[b],
            ).wait()

        def write(c, b, nrows=_CHUNK):
            pltpu.sync_copy(
                gbuf.at[b, pl.ds(0, nrows)],
                out_hbm.at[pl.ds(base + c * _CHUNK, nrows)],
            )

        for b in range(_NBUF):
            start_g(b, b)

        def outer(g, carry):
            for b in range(_NBUF):
                c = g * _NBUF + b
                wait_g(c, b)
                write(c, b)
                start_g(c + _NBUF, b)
            return carry

        n_main = ((n_chunks - _NBUF) // _NBUF) * _NBUF
        lax.fori_loop(0, n_main // _NBUF, outer, 0)

        tail_work = [(c, _CHUNK) for c in range(n_main, n_chunks)]
        if _REM:
            tail_work.append((n_chunks, _REM))
        for i, (c, nr) in enumerate(tail_work):
            b = c % _NBUF
            wait_g(c, b, nrows=nr)
            write(c, b, nrows=nr)
            if i + _NBUF < len(tail_work):
                cn, nn = tail_work[i + _NBUF]
                start_g(cn, cn % _NBUF, nrows=nn)

    return k(idx_flat, table_p)


_TBB = 1024


def _tc_transpose(x3):
    n = x3.shape[0]

    def body(x_ref, o_ref):
        for t in range(7):
            o_ref[pl.ds(t * 128, 128), :] = x_ref[:, t, :].T
        o_ref[pl.ds(896, _VOCAB - 896), :] = x_ref[:, 7, : _VOCAB - 896].T

    return pl.pallas_call(
        body,
        grid=(n // _TBB,),
        in_specs=[pl.BlockSpec((_TBB, 8, 128), lambda i: (i, 0, 0))],
        out_specs=pl.BlockSpec((_VOCAB, _TBB), lambda i: (0, i)),
        out_shape=jax.ShapeDtypeStruct((_VOCAB, n), jnp.float32),
    )(x3)


def kernel(idx, table):
    b, s = idx.shape
    idx_flat = idx.reshape(b * s).astype(jnp.int32)
    table_p = jnp.pad(table.astype(jnp.float32), ((0, 0), (0, _DPAD - _VOCAB)))
    gathered = _sc_gather(idx_flat, table_p)
    return _tc_transpose(gathered.reshape(b * s, 8, 128)).T


# speedup vs baseline: 1.0697x; 1.0697x over previous
import functools

import jax
import jax.numpy as jnp
from jax import lax
from jax.experimental import pallas as pl
from jax.experimental.pallas import tpu as pltpu
from jax.experimental.pallas import tpu_sc as plsc

_VOCAB = 1000
_DPAD = 1024
_NC = 2
_NS = 16
_NW = _NC * _NS
_CHUNK = 24
_NBUF = 2
_REM = 16


def _sc_gather(idx_flat, table_p):
    n = idx_flat.shape[0]
    b_per_w = n // _NW
    n_chunks = (b_per_w - _REM) // _CHUNK
    mesh = plsc.VectorSubcoreMesh(core_axis_name="c", subcore_axis_name="s")

    @functools.partial(
        pl.kernel,
        mesh=mesh,
        compiler_params=pltpu.CompilerParams(use_tc_tiling_on_sc=False),
        out_type=jax.ShapeDtypeStruct((n, _DPAD), jnp.float32),
        scratch_types=[
            pltpu.VMEM((b_per_w,), jnp.int32),
            pltpu.VMEM((_NBUF, _CHUNK, _DPAD), jnp.float32),
            pltpu.VMEM_SHARED((_VOCAB, _DPAD), jnp.float32),
            pltpu.SemaphoreType.DMA,
            pltpu.SemaphoreType.DMA,
        ],
    )
    def k(idx_hbm, table_hbm, out_hbm, idx_v, gbuf, table_sp, gs0, gs1):
        gsems = [gs0, gs1]
        cid = lax.axis_index("c")
        sid = lax.axis_index("s")
        wid = sid * _NC + cid
        base = wid * b_per_w

        @pl.when(sid < 15)
        def _():
            pltpu.sync_copy(
                table_hbm.at[pl.ds(sid * 64, 64)],
                table_sp.at[pl.ds(sid * 64, 64)],
            )

        @pl.when(sid == 15)
        def _():
            pltpu.sync_copy(
                table_hbm.at[pl.ds(960, _VOCAB - 960)],
                table_sp.at[pl.ds(960, _VOCAB - 960)],
            )

        pltpu.sync_copy(idx_hbm.at[pl.ds(base, b_per_w)], idx_v)
        plsc.subcore_barrier()

        def start_g(c, b, nrows=_CHUNK):
            pltpu.async_copy(
                table_sp.at[idx_v.at[pl.ds(c * _CHUNK, nrows)]],
                gbuf.at[b, pl.ds(0, nrows)],
                gsems[b],
            )

        def wait_g(c, b, nrows=_CHUNK):
            pltpu.make_async_copy(
                table_sp.at[idx_v.at[pl.ds(c * _CHUNK, nrows)]],
                gbuf.at[b, pl.ds(0, nrows)],
                gsems[b],
            ).wait()

        def write(c, b, nrows=_CHUNK):
            pltpu.sync_copy(
                gbuf.at[b, pl.ds(0, nrows)],
                out_hbm.at[pl.ds(base + c * _CHUNK, nrows)],
            )

        for b in range(_NBUF):
            start_g(b, b)

        def outer(g, carry):
            for b in range(_NBUF):
                c = g * _NBUF + b
                wait_g(c, b)
                write(c, b)
                start_g(c + _NBUF, b)
            return carry

        n_main = ((n_chunks - _NBUF) // _NBUF) * _NBUF
        lax.fori_loop(0, n_main // _NBUF, outer, 0)

        tail_work = [(c, _CHUNK) for c in range(n_main, n_chunks)]
        if _REM:
            tail_work.append((n_chunks, _REM))
        for i, (c, nr) in enumerate(tail_work):
            b = c % _NBUF
            wait_g(c, b, nrows=nr)
            write(c, b, nrows=nr)
            if i + _NBUF < len(tail_work):
                cn, nn = tail_work[i + _NBUF]
                start_g(cn, cn % _NBUF, nrows=nn)

    return k(idx_flat, table_p)


_TBB = 2560


def _tc_transpose(x3):
    n = x3.shape[0]

    def body(x_ref, o_ref):
        for t in range(7):
            o_ref[pl.ds(t * 128, 128), :] = x_ref[:, t, :].T
        o_ref[pl.ds(896, _VOCAB - 896), :] = x_ref[:, 7, : _VOCAB - 896].T

    return pl.pallas_call(
        body,
        grid=(n // _TBB,),
        in_specs=[pl.BlockSpec((_TBB, 8, 128), lambda i: (i, 0, 0))],
        out_specs=pl.BlockSpec((_VOCAB, _TBB), lambda i: (0, i)),
        out_shape=jax.ShapeDtypeStruct((_VOCAB, n), jnp.float32),
    )(x3)


def kernel(idx, table):
    b, s = idx.shape
    idx_flat = idx.reshape(b * s).astype(jnp.int32)
    table_p = jnp.pad(table.astype(jnp.float32), ((0, 0), (0, _DPAD - _VOCAB)))
    gathered = _sc_gather(idx_flat, table_p)
    return _tc_transpose(gathered.reshape(b * s, 8, 128)).T
